# Initial kernel scaffold; baseline (speedup 1.0000x reference)
import jax
import jax.numpy as jnp
from jax.experimental import pallas as pl


def kernel(h, x, edges, ca_idx, params):
    raise NotImplementedError("write your pallas kernel here")


# speedup vs baseline: 2.6688x; 2.6688x over previous
import dataclasses
import functools

import jax
import jax.numpy as jnp
from jax import lax
from jax.experimental import pallas as pl
from jax.experimental.pallas import tpu as pltpu
from jax.experimental.pallas import tpu_sc as plsc

N = 10000
E = 160000
HID = 128
EBLK = 128
NBLK = E // EBLK
NW = 32
NSUB = 16
L = 16
TBLK = 3200
SUBB = TBLK // EBLK
NTBLK = 1000
PREC = jax.lax.Precision.HIGHEST
F32 = jnp.float32


def _silu(v):
    return v / (1.0 + jnp.exp(-v))


def _dot(a, b):
    return jnp.dot(a, b, precision=PREC, preferred_element_type=F32)


def _sc_params():
    cp = pltpu.CompilerParams()
    if "needs_layout_passes" in pltpu.CompilerParams.__dataclass_fields__:
        cp = dataclasses.replace(cp, needs_layout_passes=False)
    return cp


@functools.cache
def _sc_mesh():
    return plsc.VectorSubcoreMesh(core_axis_name="c", subcore_axis_name="s",
                                  num_cores=2, num_subcores=NSUB)


@jax.jit
def _sc_gather(tab_a, tab_b, xflat, row, col):
    @functools.partial(
        pl.kernel,
        out_type=(jax.ShapeDtypeStruct((E, HID), F32),
                  jax.ShapeDtypeStruct((E, HID), F32),
                  jax.ShapeDtypeStruct((NBLK * 8, EBLK), F32)),
        mesh=_sc_mesh(),
        scratch_types=[
            pltpu.VMEM((EBLK,), jnp.int32),
            pltpu.VMEM((EBLK,), jnp.int32),
            pltpu.VMEM((EBLK, HID), F32),
            pltpu.VMEM((EBLK, HID), F32),
            pltpu.VMEM((3 * N,), F32),
            pltpu.VMEM((8, EBLK), F32),
            pltpu.SemaphoreType.DMA,
            pltpu.SemaphoreType.DMA,
            pltpu.SemaphoreType.DMA,
        ],
        compiler_params=_sc_params(),
    )
    def k(ta_hbm, tb_hbm, x_hbm, row_hbm, col_hbm, oa_hbm, ob_hbm, aux_hbm,
          idx_a, idx_b, buf_a, buf_b, xbuf, stage, sem_a, sem_b, sem_x):
        wid = lax.axis_index("s") * 2 + lax.axis_index("c")
        pltpu.async_copy(x_hbm, xbuf, sem_x).wait()

        @pl.loop(wid, NBLK, step=NW)
        def _(b):
            base = b * EBLK
            pltpu.sync_copy(row_hbm.at[pl.ds(base, EBLK)], idx_a)
            pltpu.sync_copy(col_hbm.at[pl.ds(base, EBLK)], idx_b)
            cp_a = pltpu.async_copy(ta_hbm.at[idx_a], buf_a, sem_a)
            cp_b = pltpu.async_copy(tb_hbm.at[idx_b], buf_b, sem_b)
            for j in range(EBLK // L):
                ia3 = idx_a[pl.ds(j * L, L)] * 3
                ib3 = idx_b[pl.ds(j * L, L)] * 3
                rad = jnp.zeros((L,), F32)
                for d in range(3):
                    ds = jnp.full((L,), d, jnp.int32)
                    cd = (plsc.load_gather(xbuf, [ia3 + ds])
                          - plsc.load_gather(xbuf, [ib3 + ds]))
                    stage[d, pl.ds(j * L, L)] = cd
                    rad = rad + cd * cd
                stage[3, pl.ds(j * L, L)] = rad
            pltpu.sync_copy(stage, aux_hbm.at[pl.ds(b * 8, 8)])
            cp_a.wait()
            cp_b.wait()
            pltpu.sync_copy(buf_a, oa_hbm.at[pl.ds(base, EBLK)])
            pltpu.sync_copy(buf_b, ob_hbm.at[pl.ds(base, EBLK)])

    return k(tab_a, tab_b, xflat, row, col)


@jax.jit
def _sc_scatter(m, tail, row, zeros_nh):
    @functools.partial(
        pl.kernel,
        out_type=jax.ShapeDtypeStruct((2, N, HID), F32),
        mesh=_sc_mesh(),
        scratch_types=[
            pltpu.VMEM((EBLK,), jnp.int32),
            pltpu.VMEM((EBLK, HID), F32),
            pltpu.VMEM_SHARED((N, HID), F32),
            pltpu.SemaphoreType.DMA,
        ],
        compiler_params=_sc_params(),
    )
    def k(m_hbm, tail_hbm, row_hbm, z_hbm, out_hbm, idx, buf, acc, sem):
        cid = lax.axis_index("c")
        sid = lax.axis_index("s")
        chunk = 80

        @pl.loop(sid, N // chunk, step=NSUB)
        def _(g):
            sl = pl.ds(g * chunk, chunk)
            pltpu.sync_copy(z_hbm.at[sl], acc.at[sl])

        plsc.subcore_barrier()

        @pl.when(cid == 0)
        def _():
            @pl.loop(sid, NBLK, step=NSUB)
            def _(b):
                base = b * EBLK
                pltpu.sync_copy(row_hbm.at[pl.ds(base, EBLK)], idx)
                pltpu.sync_copy(m_hbm.at[pl.ds(base, EBLK)], buf)
                pltpu.sync_copy(buf, acc.at[idx], add=True)

        @pl.when(cid == 1)
        def _():
            @pl.loop(sid, NBLK, step=NSUB)
            def _(b):
                base = b * EBLK
                pltpu.sync_copy(row_hbm.at[pl.ds(base, EBLK)], idx)
                pltpu.sync_copy(tail_hbm.at[pl.ds(base, EBLK)], buf)
                pltpu.sync_copy(buf, acc.at[idx], add=True)

        plsc.subcore_barrier()

        @pl.loop(sid, N // chunk, step=NSUB)
        def _(g):
            sl = pl.ds(g * chunk, chunk)
            pltpu.sync_copy(acc.at[sl], out_hbm.at[cid, sl])

    return k(m, tail, row, zeros_nh)


def _full(shape):
    nd = len(shape)
    return pl.BlockSpec(shape, lambda *_: (0,) * nd)


def _init_tc(h, embw, embb, waw, wab, wbw):
    def body(h_ref, ew_ref, eb_ref, aw_ref, ab_ref, bw_ref,
             h0_ref, ta_ref, tb_ref):
        h0 = _dot(h_ref[...], ew_ref[...]) + eb_ref[...]
        h0_ref[...] = h0
        ta_ref[...] = _dot(h0, aw_ref[...]) + ab_ref[...]
        tb_ref[...] = _dot(h0, bw_ref[...])

    blk = pl.BlockSpec((NTBLK, HID), lambda i: (i, 0))
    return pl.pallas_call(
        body,
        grid=(N // NTBLK,),
        in_specs=[blk, _full((HID, HID)), _full((1, HID)),
                  _full((HID, HID)), _full((1, HID)), _full((HID, HID))],
        out_specs=[blk, blk, blk],
        out_shape=[jax.ShapeDtypeStruct((N, HID), F32)] * 3,
    )(h, embw, embb, waw, wab, wbw)


def _edge_tc(ga, gb, aux, e2w, e2b, c1w, c1b, c2r, wr, e1b):
    def body(ga_ref, gb_ref, aux_ref, e2w_ref, e2b_ref, c1w_ref, c1b_ref,
             c2r_ref, wr_ref, e1b_ref, m_ref, tail_ref):
        at = jnp.concatenate(
            [jnp.transpose(aux_ref[pl.ds(s * 8, 8), :], (1, 0))
             for s in range(SUBB)], axis=0)
        cd = at[:, :3]
        radial = at[:, 3:4]
        pre = ga_ref[...] + gb_ref[...] + radial * wr_ref[...] + e1b_ref[...]
        m = _silu(_dot(_silu(pre), e2w_ref[...]) + e2b_ref[...])
        t2 = _silu(_dot(m, c1w_ref[...]) + c1b_ref[...])
        t = jnp.sum(t2 * c2r_ref[...], axis=1, keepdims=True)
        m_ref[...] = m
        tail_ref[...] = jnp.concatenate(
            [cd * t, jnp.ones((TBLK, 1), F32),
             jnp.zeros((TBLK, HID - 4), F32)], axis=1)

    eblk = pl.BlockSpec((TBLK, HID), lambda i: (i, 0))
    return pl.pallas_call(
        body,
        grid=(E // TBLK,),
        in_specs=[
            eblk, eblk,
            pl.BlockSpec((SUBB * 8, EBLK), lambda i: (i, 0)),
            _full((HID, HID)), _full((1, HID)),
            _full((HID, HID)), _full((1, HID)),
            _full((1, HID)), _full((1, HID)), _full((1, HID)),
        ],
        out_specs=[eblk, eblk],
        out_shape=[jax.ShapeDtypeStruct((E, HID), F32)] * 2,
    )(ga, gb, aux, e2w, e2b, c1w, c1b, c2r, wr, e1b)


def _node_tc(h, x, agg, tl, n1aw, n1bw, n1b, n2w, n2b, naw, nab, nbw):
    def body(h_ref, x_ref, agg_ref, tl_ref, n1aw_ref, n1bw_ref, n1b_ref,
             n2w_ref, n2b_ref, naw_ref, nab_ref, nbw_ref,
             ho_ref, xo_ref, ta_ref, tb_ref):
        tsum = tl_ref[...][:, :3]
        cnt = tl_ref[...][:, 3:4]
        xo_ref[...] = x_ref[...] + tsum / jnp.maximum(cnt, 1.0)
        h_in = h_ref[...]
        z = _silu(_dot(h_in, n1aw_ref[...]) + _dot(agg_ref[...], n1bw_ref[...])
                  + n1b_ref[...])
        hn = h_in + _dot(z, n2w_ref[...]) + n2b_ref[...]
        ho_ref[...] = hn
        ta_ref[...] = _dot(hn, naw_ref[...]) + nab_ref[...]
        tb_ref[...] = _dot(hn, nbw_ref[...])

    blk = pl.BlockSpec((NTBLK, HID), lambda i: (i, 0))
    blk3 = pl.BlockSpec((NTBLK, 3), lambda i: (i, 0))
    return pl.pallas_call(
        body,
        grid=(N // NTBLK,),
        in_specs=[
            blk, blk3, blk, blk,
            _full((HID, HID)), _full((HID, HID)), _full((1, HID)),
            _full((HID, HID)), _full((1, HID)),
            _full((HID, HID)), _full((1, HID)), _full((HID, HID)),
        ],
        out_specs=[blk, blk3, blk, blk],
        out_shape=[
            jax.ShapeDtypeStruct((N, HID), F32),
            jax.ShapeDtypeStruct((N, 3), F32),
            jax.ShapeDtypeStruct((N, HID), F32),
            jax.ShapeDtypeStruct((N, HID), F32),
        ],
    )(h, x, agg, tl, n1aw, n1bw, n1b, n2w, n2b, naw, nab, nbw)


def _head_tc(tab, ca2, l1w, l1b, l2w, l2b):
    ng = ca2.shape[0]

    def body(tab_ref, ca_ref, l1w_ref, l1b_ref, l2w_ref, l2b_ref, out_ref):
        iota = lax.broadcasted_iota(jnp.int32, (ng, N), 1)
        oh = (iota == ca_ref[...]).astype(F32)
        ch = _dot(oh, tab_ref[...])
        y = jnp.maximum(_dot(ch, l1w_ref[...]) + l1b_ref[...], 0.0)
        out_ref[...] = _dot(y, l2w_ref[...]) + l2b_ref[...]

    return pl.pallas_call(
        body,
        in_specs=[
            _full((N, HID)), _full((ng, 1)),
            _full((HID, 64)), _full((1, 64)),
            _full((64, HID)), _full((1, HID)),
        ],
        out_specs=_full((ng, HID)),
        out_shape=jax.ShapeDtypeStruct((ng, HID), F32),
    )(tab, ca2, l1w, l1b, l2w, l2b)


def kernel(h, x, edges, ca_idx, params):
    row = edges[0]
    col = edges[1]
    zeros_nh = jnp.zeros((N, HID), F32)
    zeros_hh = jnp.zeros((HID, HID), F32)

    def r1(v):
        return v.reshape(1, -1)

    lps = params["layers"]

    def proj_w(lp):
        e1w = lp["e1"]["w"]
        return e1w[:HID], r1(lp["e1"]["b"]), e1w[HID:2 * HID], r1(e1w[2 * HID])

    wa0, ab0, wb0, _ = proj_w(lps[0])
    hcur, ta, tb = _init_tc(h, params["emb_in"]["w"], r1(params["emb_in"]["b"]),
                            wa0, ab0, wb0)
    xcur = x
    for li, lp in enumerate(lps):
        _, _, _, wr = proj_w(lp)
        ga, gb, aux = _sc_gather(ta, tb, xcur.reshape(-1), row, col)
        m, tail = _edge_tc(ga, gb, aux,
                           lp["e2"]["w"], r1(lp["e2"]["b"]),
                           lp["c1"]["w"], r1(lp["c1"]["b"]),
                           r1(lp["c2"]["w"]), wr, r1(lp["e1"]["b"]))
        parts = _sc_scatter(m, tail, row, zeros_nh)
        if li + 1 < len(lps):
            naw, nab, nbw, _ = proj_w(lps[li + 1])
        else:
            naw, nab, nbw = (params["emb_out"]["w"],
                             r1(params["emb_out"]["b"]), zeros_hh)
        n1w = lp["n1"]["w"]
        hcur, xcur, ta, tb = _node_tc(
            hcur, xcur, parts[0], parts[1],
            n1w[:HID], n1w[HID:], r1(lp["n1"]["b"]),
            lp["n2"]["w"], r1(lp["n2"]["b"]),
            naw, nab, nbw)
    return _head_tc(ta, ca_idx.reshape(-1, 1).astype(jnp.int32),
                    params["mlp_l1"]["w"], r1(params["mlp_l1"]["b"]),
                    params["mlp_l2"]["w"], r1(params["mlp_l2"]["b"]))


# speedup vs baseline: 2.7259x; 1.0214x over previous
import dataclasses
import functools

import jax
import jax.numpy as jnp
from jax import lax
from jax.experimental import pallas as pl
from jax.experimental.pallas import tpu as pltpu
from jax.experimental.pallas import tpu_sc as plsc

N = 10000
E = 160000
HID = 128
EBLK = 128
NBLK = E // EBLK
NW = 32
NSUB = 16
L = 16
TBLK = 3200
SUBB = TBLK // EBLK
NTBLK = 1000
PREC = jax.lax.Precision.HIGHEST
F32 = jnp.float32


def _silu(v):
    return v / (1.0 + jnp.exp(-v))


def _dot(a, b):
    return jnp.dot(a, b, precision=PREC, preferred_element_type=F32)


def _sc_params():
    cp = pltpu.CompilerParams()
    if "needs_layout_passes" in pltpu.CompilerParams.__dataclass_fields__:
        cp = dataclasses.replace(cp, needs_layout_passes=False)
    return cp


@functools.cache
def _sc_mesh():
    return plsc.VectorSubcoreMesh(core_axis_name="c", subcore_axis_name="s",
                                  num_cores=2, num_subcores=NSUB)


@jax.jit
def _sc_gather(tab_a, tab_b, xflat, row, col):
    @functools.partial(
        pl.kernel,
        out_type=(jax.ShapeDtypeStruct((E, HID), F32),
                  jax.ShapeDtypeStruct((E, HID), F32),
                  jax.ShapeDtypeStruct((NBLK * 8, EBLK), F32)),
        mesh=_sc_mesh(),
        scratch_types=[
            pltpu.VMEM((EBLK,), jnp.int32),
            pltpu.VMEM((EBLK,), jnp.int32),
            pltpu.VMEM((EBLK, HID), F32),
            pltpu.VMEM((EBLK, HID), F32),
            pltpu.VMEM((3 * N,), F32),
            pltpu.VMEM((8, EBLK), F32),
            pltpu.SemaphoreType.DMA,
            pltpu.SemaphoreType.DMA,
            pltpu.SemaphoreType.DMA,
        ],
        compiler_params=_sc_params(),
    )
    def k(ta_hbm, tb_hbm, x_hbm, row_hbm, col_hbm, oa_hbm, ob_hbm, aux_hbm,
          idx_a, idx_b, buf_a, buf_b, xbuf, stage, sem_a, sem_b, sem_x):
        wid = lax.axis_index("s") * 2 + lax.axis_index("c")
        pltpu.async_copy(x_hbm, xbuf, sem_x).wait()

        @pl.loop(wid, NBLK, step=NW)
        def _(b):
            base = b * EBLK
            pltpu.sync_copy(row_hbm.at[pl.ds(base, EBLK)], idx_a)
            pltpu.sync_copy(col_hbm.at[pl.ds(base, EBLK)], idx_b)
            cp_a = pltpu.async_copy(ta_hbm.at[idx_a], buf_a, sem_a)
            cp_b = pltpu.async_copy(tb_hbm.at[idx_b], buf_b, sem_b)
            for j in range(EBLK // L):
                ia3 = idx_a[pl.ds(j * L, L)] * 3
                ib3 = idx_b[pl.ds(j * L, L)] * 3
                rad = jnp.zeros((L,), F32)
                for d in range(3):
                    ds = jnp.full((L,), d, jnp.int32)
                    cd = (plsc.load_gather(xbuf, [ia3 + ds])
                          - plsc.load_gather(xbuf, [ib3 + ds]))
                    stage[d, pl.ds(j * L, L)] = cd
                    rad = rad + cd * cd
                stage[3, pl.ds(j * L, L)] = rad
            pltpu.sync_copy(stage, aux_hbm.at[pl.ds(b * 8, 8)])
            cp_a.wait()
            cp_b.wait()
            pltpu.sync_copy(buf_a, oa_hbm.at[pl.ds(base, EBLK)])
            pltpu.sync_copy(buf_b, ob_hbm.at[pl.ds(base, EBLK)])

    return k(tab_a, tab_b, xflat, row, col)


@jax.jit
def _sc_scatter(m, tail, row, zeros_nh):
    @functools.partial(
        pl.kernel,
        out_type=jax.ShapeDtypeStruct((2, N, HID), F32),
        mesh=_sc_mesh(),
        scratch_types=[
            pltpu.VMEM((EBLK,), jnp.int32),
            pltpu.VMEM((EBLK, HID), F32),
            pltpu.VMEM_SHARED((N, HID), F32),
        ],
        compiler_params=_sc_params(),
    )
    def k(m_hbm, tail_hbm, row_hbm, z_hbm, out_hbm, idx, buf, acc):
        cid = lax.axis_index("c")
        sid = lax.axis_index("s")
        chunk = 80

        @pl.loop(sid, N // chunk, step=NSUB)
        def _(g):
            sl = pl.ds(g * chunk, chunk)
            pltpu.sync_copy(z_hbm.at[sl], acc.at[sl])

        plsc.subcore_barrier()

        @pl.when(cid == 0)
        def _():
            @pl.loop(sid, NBLK, step=NSUB)
            def _(b):
                base = b * EBLK
                pltpu.sync_copy(row_hbm.at[pl.ds(base, EBLK)], idx)
                pltpu.sync_copy(m_hbm.at[pl.ds(base, EBLK)], buf)
                pltpu.sync_copy(buf, acc.at[idx], add=True)

        @pl.when(cid == 1)
        def _():
            @pl.loop(sid, NBLK, step=NSUB)
            def _(b):
                base = b * EBLK
                pltpu.sync_copy(row_hbm.at[pl.ds(base, EBLK)], idx)
                pltpu.sync_copy(tail_hbm.at[pl.ds(base, EBLK)], buf)
                pltpu.sync_copy(buf, acc.at[idx], add=True)

        plsc.subcore_barrier()

        @pl.loop(sid, N // chunk, step=NSUB)
        def _(g):
            sl = pl.ds(g * chunk, chunk)
            pltpu.sync_copy(acc.at[sl], out_hbm.at[cid, sl])

    return k(m, tail, row, zeros_nh)


def _full(shape):
    nd = len(shape)
    return pl.BlockSpec(shape, lambda *_: (0,) * nd)


def _init_tc(h, embw, embb, waw, wab, wbw):
    def body(h_ref, ew_ref, eb_ref, aw_ref, ab_ref, bw_ref,
             h0_ref, ta_ref, tb_ref):
        h0 = _dot(h_ref[...], ew_ref[...]) + eb_ref[...]
        h0_ref[...] = h0
        ta_ref[...] = _dot(h0, aw_ref[...]) + ab_ref[...]
        tb_ref[...] = _dot(h0, bw_ref[...])

    blk = pl.BlockSpec((NTBLK, HID), lambda i: (i, 0))
    return pl.pallas_call(
        body,
        grid=(N // NTBLK,),
        in_specs=[blk, _full((HID, HID)), _full((1, HID)),
                  _full((HID, HID)), _full((1, HID)), _full((HID, HID))],
        out_specs=[blk, blk, blk],
        out_shape=[jax.ShapeDtypeStruct((N, HID), F32)] * 3,
    )(h, embw, embb, waw, wab, wbw)


def _edge_tc(ga, gb, aux, e2w, e2b, c1w, c1b, c2r, wr):
    def body(ga_ref, gb_ref, aux_ref, e2w_ref, e2b_ref, c1w_ref, c1b_ref,
             c2r_ref, wr_ref, m_ref, tail_ref):
        a3 = aux_ref[...].reshape(SUBB, 8, EBLK)
        li = lax.broadcasted_iota(jnp.int32, (TBLK, EBLK), 1)
        ri = lax.broadcasted_iota(jnp.int32, (TBLK, EBLK), 0)
        dmask = (li == ri % EBLK).astype(F32)

        def col(d):
            b = jnp.broadcast_to(a3[:, d:d + 1, :],
                                 (SUBB, EBLK, EBLK)).reshape(TBLK, EBLK)
            return jnp.sum(b * dmask, axis=1, keepdims=True)

        cd0, cd1, cd2, radial = col(0), col(1), col(2), col(3)
        pre = ga_ref[...] + gb_ref[...] + radial * wr_ref[...]
        m = _silu(_dot(_silu(pre), e2w_ref[...]) + e2b_ref[...])
        t2 = _silu(_dot(m, c1w_ref[...]) + c1b_ref[...])
        t = jnp.sum(t2 * c2r_ref[...], axis=1, keepdims=True)
        m_ref[...] = m
        tail_ref[...] = jnp.concatenate(
            [cd0 * t, cd1 * t, cd2 * t, jnp.ones((TBLK, 1), F32),
             jnp.zeros((TBLK, HID - 4), F32)], axis=1)

    eblk = pl.BlockSpec((TBLK, HID), lambda i: (i, 0))
    return pl.pallas_call(
        body,
        grid=(E // TBLK,),
        in_specs=[
            eblk, eblk,
            pl.BlockSpec((SUBB * 8, EBLK), lambda i: (i, 0)),
            _full((HID, HID)), _full((1, HID)),
            _full((HID, HID)), _full((1, HID)),
            _full((1, HID)), _full((1, HID)),
        ],
        out_specs=[eblk, eblk],
        out_shape=[jax.ShapeDtypeStruct((E, HID), F32)] * 2,
    )(ga, gb, aux, e2w, e2b, c1w, c1b, c2r, wr)


def _node_tc(h, x, parts, n1aw, n1bw, n1b, n2w, n2b, naw, nab, nbw):
    def body(h_ref, x_ref, p_ref, n1aw_ref, n1bw_ref, n1b_ref,
             n2w_ref, n2b_ref, naw_ref, nab_ref, nbw_ref,
             ho_ref, xo_ref, ta_ref, tb_ref):
        tlv = p_ref[1]
        tsum = tlv[:, :3]
        cnt = tlv[:, 3:4]
        xo_ref[...] = x_ref[...] + tsum / jnp.maximum(cnt, 1.0)
        agg = p_ref[0]
        h_in = h_ref[...]
        z = _silu(_dot(h_in, n1aw_ref[...]) + _dot(agg, n1bw_ref[...])
                  + n1b_ref[...])
        hn = h_in + _dot(z, n2w_ref[...]) + n2b_ref[...]
        ho_ref[...] = hn
        ta_ref[...] = _dot(hn, naw_ref[...]) + nab_ref[...]
        tb_ref[...] = _dot(hn, nbw_ref[...])

    blk = pl.BlockSpec((NTBLK, HID), lambda i: (i, 0))
    blk3 = pl.BlockSpec((NTBLK, 3), lambda i: (i, 0))
    pblk = pl.BlockSpec((2, NTBLK, HID), lambda i: (0, i, 0))
    return pl.pallas_call(
        body,
        grid=(N // NTBLK,),
        in_specs=[
            blk, blk3, pblk,
            _full((HID, HID)), _full((HID, HID)), _full((1, HID)),
            _full((HID, HID)), _full((1, HID)),
            _full((HID, HID)), _full((1, HID)), _full((HID, HID)),
        ],
        out_specs=[blk, blk3, blk, blk],
        out_shape=[
            jax.ShapeDtypeStruct((N, HID), F32),
            jax.ShapeDtypeStruct((N, 3), F32),
            jax.ShapeDtypeStruct((N, HID), F32),
            jax.ShapeDtypeStruct((N, HID), F32),
        ],
    )(h, x, parts, n1aw, n1bw, n1b, n2w, n2b, naw, nab, nbw)


def _head_tc(tab, ca2, l1w, l1b, l2w, l2b):
    ng = ca2.shape[0]

    def body(tab_ref, ca_ref, l1w_ref, l1b_ref, l2w_ref, l2b_ref, out_ref):
        iota = lax.broadcasted_iota(jnp.int32, (ng, N), 1)
        oh = (iota == ca_ref[...]).astype(F32)
        ch = _dot(oh, tab_ref[...])
        y = jnp.maximum(_dot(ch, l1w_ref[...]) + l1b_ref[...], 0.0)
        out_ref[...] = _dot(y, l2w_ref[...]) + l2b_ref[...]

    return pl.pallas_call(
        body,
        in_specs=[
            _full((N, HID)), _full((ng, 1)),
            _full((HID, 64)), _full((1, 64)),
            _full((64, HID)), _full((1, HID)),
        ],
        out_specs=_full((ng, HID)),
        out_shape=jax.ShapeDtypeStruct((ng, HID), F32),
    )(tab, ca2, l1w, l1b, l2w, l2b)


def kernel(h, x, edges, ca_idx, params):
    row = edges[0]
    col = edges[1]
    zeros_nh = jnp.zeros((N, HID), F32)
    zeros_hh = jnp.zeros((HID, HID), F32)

    def r1(v):
        return v.reshape(1, -1)

    lps = params["layers"]

    def proj_w(lp):
        e1w = lp["e1"]["w"]
        return e1w[:HID], r1(lp["e1"]["b"]), e1w[HID:2 * HID], r1(e1w[2 * HID])

    wa0, ab0, wb0, _ = proj_w(lps[0])
    hcur, ta, tb = _init_tc(h, params["emb_in"]["w"], r1(params["emb_in"]["b"]),
                            wa0, ab0, wb0)
    xcur = x
    for li, lp in enumerate(lps):
        _, _, _, wr = proj_w(lp)
        ga, gb, aux = _sc_gather(ta, tb, xcur.reshape(-1), row, col)
        m, tail = _edge_tc(ga, gb, aux,
                           lp["e2"]["w"], r1(lp["e2"]["b"]),
                           lp["c1"]["w"], r1(lp["c1"]["b"]),
                           r1(lp["c2"]["w"]), wr)
        parts = _sc_scatter(m, tail, row, zeros_nh)
        if li + 1 < len(lps):
            naw, nab, nbw, _ = proj_w(lps[li + 1])
        else:
            naw, nab, nbw = (params["emb_out"]["w"],
                             r1(params["emb_out"]["b"]), zeros_hh)
        n1w = lp["n1"]["w"]
        hcur, xcur, ta, tb = _node_tc(
            hcur, xcur, parts,
            n1w[:HID], n1w[HID:], r1(lp["n1"]["b"]),
            lp["n2"]["w"], r1(lp["n2"]["b"]),
            naw, nab, nbw)
    return _head_tc(ta, ca_idx.reshape(-1, 1).astype(jnp.int32),
                    params["mlp_l1"]["w"], r1(params["mlp_l1"]["b"]),
                    params["mlp_l2"]["w"], r1(params["mlp_l2"]["b"]))


# speedup vs baseline: 3.2902x; 1.2070x over previous
import dataclasses
import functools

import jax
import jax.numpy as jnp
from jax import lax
from jax.experimental import pallas as pl
from jax.experimental.pallas import tpu as pltpu
from jax.experimental.pallas import tpu_sc as plsc

N = 10000
E = 160000
HID = 128
EBLK = 128
NBLK = E // EBLK
NW = 32
NSUB = 16
L = 16
TBLK = 3200
SUBB = TBLK // EBLK
NTBLK = 1000
PREC = jax.lax.Precision.HIGHEST
F32 = jnp.float32


def _silu(v):
    return v / (1.0 + jnp.exp(-v))


def _dot(a, b):
    return jnp.dot(a, b, precision=PREC, preferred_element_type=F32)


def _dot_h(a, b):
    bf16 = jnp.bfloat16
    ah = a.astype(bf16)
    al = (a - ah.astype(F32)).astype(bf16)
    bh = b.astype(bf16)
    bl = (b - bh.astype(F32)).astype(bf16)

    def d(u, v):
        return jnp.dot(u, v, preferred_element_type=F32)

    return d(ah, bl) + d(al, bh) + d(ah, bh)


def _sc_params():
    cp = pltpu.CompilerParams()
    if "needs_layout_passes" in pltpu.CompilerParams.__dataclass_fields__:
        cp = dataclasses.replace(cp, needs_layout_passes=False)
    return cp


@functools.cache
def _sc_mesh():
    return plsc.VectorSubcoreMesh(core_axis_name="c", subcore_axis_name="s",
                                  num_cores=2, num_subcores=NSUB)


@jax.jit
def _sc_gather(tab_a, tab_b, xflat, row, col):
    @functools.partial(
        pl.kernel,
        out_type=(jax.ShapeDtypeStruct((E, HID), F32),
                  jax.ShapeDtypeStruct((E, HID), F32),
                  jax.ShapeDtypeStruct((NBLK * 8, EBLK), F32)),
        mesh=_sc_mesh(),
        scratch_types=[
            pltpu.VMEM((EBLK,), jnp.int32),
            pltpu.VMEM((EBLK,), jnp.int32),
            pltpu.VMEM((EBLK, HID), F32),
            pltpu.VMEM((EBLK, HID), F32),
            pltpu.VMEM((3 * N,), F32),
            pltpu.VMEM((8, EBLK), F32),
            pltpu.SemaphoreType.DMA,
            pltpu.SemaphoreType.DMA,
            pltpu.SemaphoreType.DMA,
        ],
        compiler_params=_sc_params(),
    )
    def k(ta_hbm, tb_hbm, x_hbm, row_hbm, col_hbm, oa_hbm, ob_hbm, aux_hbm,
          idx_a, idx_b, buf_a, buf_b, xbuf, stage, sem_a, sem_b, sem_x):
        wid = lax.axis_index("s") * 2 + lax.axis_index("c")
        pltpu.async_copy(x_hbm, xbuf, sem_x).wait()

        @pl.loop(wid, NBLK, step=NW)
        def _(b):
            base = b * EBLK
            pltpu.sync_copy(row_hbm.at[pl.ds(base, EBLK)], idx_a)
            pltpu.sync_copy(col_hbm.at[pl.ds(base, EBLK)], idx_b)
            cp_a = pltpu.async_copy(ta_hbm.at[idx_a], buf_a, sem_a)
            cp_b = pltpu.async_copy(tb_hbm.at[idx_b], buf_b, sem_b)
            for j in range(EBLK // L):
                ia3 = idx_a[pl.ds(j * L, L)] * 3
                ib3 = idx_b[pl.ds(j * L, L)] * 3
                rad = jnp.zeros((L,), F32)
                for d in range(3):
                    ds = jnp.full((L,), d, jnp.int32)
                    cd = (plsc.load_gather(xbuf, [ia3 + ds])
                          - plsc.load_gather(xbuf, [ib3 + ds]))
                    stage[d, pl.ds(j * L, L)] = cd
                    rad = rad + cd * cd
                stage[3, pl.ds(j * L, L)] = rad
            pltpu.sync_copy(stage, aux_hbm.at[pl.ds(b * 8, 8)])
            cp_a.wait()
            cp_b.wait()
            pltpu.sync_copy(buf_a, oa_hbm.at[pl.ds(base, EBLK)])
            pltpu.sync_copy(buf_b, ob_hbm.at[pl.ds(base, EBLK)])

    return k(tab_a, tab_b, xflat, row, col)


@jax.jit
def _sc_scatter(m, tail, row, zeros_nh):
    @functools.partial(
        pl.kernel,
        out_type=jax.ShapeDtypeStruct((2, N, HID), F32),
        mesh=_sc_mesh(),
        scratch_types=[
            pltpu.VMEM((EBLK,), jnp.int32),
            pltpu.VMEM((EBLK, HID), F32),
            pltpu.VMEM_SHARED((N, HID), F32),
        ],
        compiler_params=_sc_params(),
    )
    def k(m_hbm, tail_hbm, row_hbm, z_hbm, out_hbm, idx, buf, acc):
        cid = lax.axis_index("c")
        sid = lax.axis_index("s")
        chunk = 80

        @pl.loop(sid, N // chunk, step=NSUB)
        def _(g):
            sl = pl.ds(g * chunk, chunk)
            pltpu.sync_copy(z_hbm.at[sl], acc.at[sl])

        plsc.subcore_barrier()

        @pl.when(cid == 0)
        def _():
            @pl.loop(sid, NBLK, step=NSUB)
            def _(b):
                base = b * EBLK
                pltpu.sync_copy(row_hbm.at[pl.ds(base, EBLK)], idx)
                pltpu.sync_copy(m_hbm.at[pl.ds(base, EBLK)], buf)
                pltpu.sync_copy(buf, acc.at[idx], add=True)

        @pl.when(cid == 1)
        def _():
            @pl.loop(sid, NBLK, step=NSUB)
            def _(b):
                base = b * EBLK
                pltpu.sync_copy(row_hbm.at[pl.ds(base, EBLK)], idx)
                pltpu.sync_copy(tail_hbm.at[pl.ds(base, EBLK)], buf)
                pltpu.sync_copy(buf, acc.at[idx], add=True)

        plsc.subcore_barrier()

        @pl.loop(sid, N // chunk, step=NSUB)
        def _(g):
            sl = pl.ds(g * chunk, chunk)
            pltpu.sync_copy(acc.at[sl], out_hbm.at[cid, sl])

    return k(m, tail, row, zeros_nh)


def _full(shape):
    nd = len(shape)
    return pl.BlockSpec(shape, lambda *_: (0,) * nd)


def _init_tc(h, embw, embb, waw, wab, wbw):
    def body(h_ref, ew_ref, eb_ref, aw_ref, ab_ref, bw_ref,
             h0_ref, ta_ref, tb_ref):
        h0 = _dot(h_ref[...], ew_ref[...]) + eb_ref[...]
        h0_ref[...] = h0
        ta_ref[...] = _dot(h0, aw_ref[...]) + ab_ref[...]
        tb_ref[...] = _dot(h0, bw_ref[...])

    blk = pl.BlockSpec((NTBLK, HID), lambda i: (i, 0))
    return pl.pallas_call(
        body,
        grid=(N // NTBLK,),
        in_specs=[blk, _full((HID, HID)), _full((1, HID)),
                  _full((HID, HID)), _full((1, HID)), _full((HID, HID))],
        out_specs=[blk, blk, blk],
        out_shape=[jax.ShapeDtypeStruct((N, HID), F32)] * 3,
    )(h, embw, embb, waw, wab, wbw)


def _edge_tc(ga, gb, aux, e2w, e2b, c1w, c1b, c2r, wr):
    def body(ga_ref, gb_ref, aux_ref, e2w_ref, e2b_ref, c1w_ref, c1b_ref,
             c2r_ref, wr_ref, m_ref, tail_ref):
        a3 = aux_ref[...].reshape(SUBB, 8, EBLK)
        li = lax.broadcasted_iota(jnp.int32, (TBLK, EBLK), 1)
        ri = lax.broadcasted_iota(jnp.int32, (TBLK, EBLK), 0)
        dmask = (li == ri % EBLK).astype(F32)

        def col(d):
            b = jnp.broadcast_to(a3[:, d:d + 1, :],
                                 (SUBB, EBLK, EBLK)).reshape(TBLK, EBLK)
            return jnp.sum(b * dmask, axis=1, keepdims=True)

        cd0, cd1, cd2, radial = col(0), col(1), col(2), col(3)
        pre = ga_ref[...] + gb_ref[...] + radial * wr_ref[...]
        m = _silu(_dot_h(_silu(pre), e2w_ref[...]) + e2b_ref[...])
        t2 = _silu(_dot_h(m, c1w_ref[...]) + c1b_ref[...])
        t = jnp.sum(t2 * c2r_ref[...], axis=1, keepdims=True)
        m_ref[...] = m
        tail_ref[...] = jnp.concatenate(
            [cd0 * t, cd1 * t, cd2 * t, jnp.ones((TBLK, 1), F32),
             jnp.zeros((TBLK, HID - 4), F32)], axis=1)

    eblk = pl.BlockSpec((TBLK, HID), lambda i: (i, 0))
    return pl.pallas_call(
        body,
        grid=(E // TBLK,),
        in_specs=[
            eblk, eblk,
            pl.BlockSpec((SUBB * 8, EBLK), lambda i: (i, 0)),
            _full((HID, HID)), _full((1, HID)),
            _full((HID, HID)), _full((1, HID)),
            _full((1, HID)), _full((1, HID)),
        ],
        out_specs=[eblk, eblk],
        out_shape=[jax.ShapeDtypeStruct((E, HID), F32)] * 2,
    )(ga, gb, aux, e2w, e2b, c1w, c1b, c2r, wr)


def _node_tc(h, x, parts, n1aw, n1bw, n1b, n2w, n2b, naw, nab, nbw):
    def body(h_ref, x_ref, p_ref, n1aw_ref, n1bw_ref, n1b_ref,
             n2w_ref, n2b_ref, naw_ref, nab_ref, nbw_ref,
             ho_ref, xo_ref, ta_ref, tb_ref):
        tlv = p_ref[1]
        tsum = tlv[:, :3]
        cnt = tlv[:, 3:4]
        xo_ref[...] = x_ref[...] + tsum / jnp.maximum(cnt, 1.0)
        agg = p_ref[0]
        h_in = h_ref[...]
        z = _silu(_dot(h_in, n1aw_ref[...]) + _dot(agg, n1bw_ref[...])
                  + n1b_ref[...])
        hn = h_in + _dot(z, n2w_ref[...]) + n2b_ref[...]
        ho_ref[...] = hn
        ta_ref[...] = _dot(hn, naw_ref[...]) + nab_ref[...]
        tb_ref[...] = _dot(hn, nbw_ref[...])

    blk = pl.BlockSpec((NTBLK, HID), lambda i: (i, 0))
    blk3 = pl.BlockSpec((NTBLK, 3), lambda i: (i, 0))
    pblk = pl.BlockSpec((2, NTBLK, HID), lambda i: (0, i, 0))
    return pl.pallas_call(
        body,
        grid=(N // NTBLK,),
        in_specs=[
            blk, blk3, pblk,
            _full((HID, HID)), _full((HID, HID)), _full((1, HID)),
            _full((HID, HID)), _full((1, HID)),
            _full((HID, HID)), _full((1, HID)), _full((HID, HID)),
        ],
        out_specs=[blk, blk3, blk, blk],
        out_shape=[
            jax.ShapeDtypeStruct((N, HID), F32),
            jax.ShapeDtypeStruct((N, 3), F32),
            jax.ShapeDtypeStruct((N, HID), F32),
            jax.ShapeDtypeStruct((N, HID), F32),
        ],
    )(h, x, parts, n1aw, n1bw, n1b, n2w, n2b, naw, nab, nbw)


def _head_tc(tab, ca2, l1w, l1b, l2w, l2b):
    ng = ca2.shape[0]

    def body(tab_ref, ca_ref, l1w_ref, l1b_ref, l2w_ref, l2b_ref, out_ref):
        iota = lax.broadcasted_iota(jnp.int32, (ng, N), 1)
        oh = (iota == ca_ref[...]).astype(F32)
        ch = _dot(oh, tab_ref[...])
        y = jnp.maximum(_dot(ch, l1w_ref[...]) + l1b_ref[...], 0.0)
        out_ref[...] = _dot(y, l2w_ref[...]) + l2b_ref[...]

    return pl.pallas_call(
        body,
        in_specs=[
            _full((N, HID)), _full((ng, 1)),
            _full((HID, 64)), _full((1, 64)),
            _full((64, HID)), _full((1, HID)),
        ],
        out_specs=_full((ng, HID)),
        out_shape=jax.ShapeDtypeStruct((ng, HID), F32),
    )(tab, ca2, l1w, l1b, l2w, l2b)


def kernel(h, x, edges, ca_idx, params):
    row = edges[0]
    col = edges[1]
    zeros_nh = jnp.zeros((N, HID), F32)
    zeros_hh = jnp.zeros((HID, HID), F32)

    def r1(v):
        return v.reshape(1, -1)

    lps = params["layers"]

    def proj_w(lp):
        e1w = lp["e1"]["w"]
        return e1w[:HID], r1(lp["e1"]["b"]), e1w[HID:2 * HID], r1(e1w[2 * HID])

    wa0, ab0, wb0, _ = proj_w(lps[0])
    hcur, ta, tb = _init_tc(h, params["emb_in"]["w"], r1(params["emb_in"]["b"]),
                            wa0, ab0, wb0)
    xcur = x
    for li, lp in enumerate(lps):
        _, _, _, wr = proj_w(lp)
        ga, gb, aux = _sc_gather(ta, tb, xcur.reshape(-1), row, col)
        m, tail = _edge_tc(ga, gb, aux,
                           lp["e2"]["w"], r1(lp["e2"]["b"]),
                           lp["c1"]["w"], r1(lp["c1"]["b"]),
                           r1(lp["c2"]["w"]), wr)
        parts = _sc_scatter(m, tail, row, zeros_nh)
        if li + 1 < len(lps):
            naw, nab, nbw, _ = proj_w(lps[li + 1])
        else:
            naw, nab, nbw = (params["emb_out"]["w"],
                             r1(params["emb_out"]["b"]), zeros_hh)
        n1w = lp["n1"]["w"]
        hcur, xcur, ta, tb = _node_tc(
            hcur, xcur, parts,
            n1w[:HID], n1w[HID:], r1(lp["n1"]["b"]),
            lp["n2"]["w"], r1(lp["n2"]["b"]),
            naw, nab, nbw)
    return _head_tc(ta, ca_idx.reshape(-1, 1).astype(jnp.int32),
                    params["mlp_l1"]["w"], r1(params["mlp_l1"]["b"]),
                    params["mlp_l2"]["w"], r1(params["mlp_l2"]["b"]))


# speedup vs baseline: 3.8817x; 1.1798x over previous
import dataclasses
import functools

import jax
import jax.numpy as jnp
from jax import lax
from jax.experimental import pallas as pl
from jax.experimental.pallas import tpu as pltpu
from jax.experimental.pallas import tpu_sc as plsc

N = 10000
E = 160000
HID = 128
EBLK = 128
NBLK = E // EBLK
NW = 32
NSUB = 16
L = 16
TBLK = 3200
SUBB = TBLK // EBLK
NTBLK = 1000
PREC = jax.lax.Precision.HIGHEST
F32 = jnp.float32


def _silu(v):
    return v / (1.0 + jnp.exp(-v))


def _dot(a, b):
    return jnp.dot(a, b, precision=PREC, preferred_element_type=F32)


def _dot_h(a, b):
    bf16 = jnp.bfloat16
    ah = a.astype(bf16)
    al = (a - ah.astype(F32)).astype(bf16)
    bh = b.astype(bf16)
    bl = (b - bh.astype(F32)).astype(bf16)

    def d(u, v):
        return jnp.dot(u, v, preferred_element_type=F32)

    return d(ah, bl) + d(al, bh) + d(ah, bh)


def _sc_params():
    cp = pltpu.CompilerParams()
    if "needs_layout_passes" in pltpu.CompilerParams.__dataclass_fields__:
        cp = dataclasses.replace(cp, needs_layout_passes=False)
    return cp


@functools.cache
def _sc_mesh():
    return plsc.VectorSubcoreMesh(core_axis_name="c", subcore_axis_name="s",
                                  num_cores=2, num_subcores=NSUB)


@jax.jit
def _sc_gather(tab_a, tab_b, xflat, row, col):
    ec = row.shape[0]
    nblk = ec // EBLK

    @functools.partial(
        pl.kernel,
        out_type=(jax.ShapeDtypeStruct((ec, HID), F32),
                  jax.ShapeDtypeStruct((ec, HID), F32),
                  jax.ShapeDtypeStruct((nblk * 8, EBLK), F32)),
        mesh=_sc_mesh(),
        scratch_types=[
            pltpu.VMEM((EBLK,), jnp.int32),
            pltpu.VMEM((EBLK,), jnp.int32),
            pltpu.VMEM((EBLK, HID), F32),
            pltpu.VMEM((EBLK, HID), F32),
            pltpu.VMEM((3 * N,), F32),
            pltpu.VMEM((8, EBLK), F32),
            pltpu.SemaphoreType.DMA,
            pltpu.SemaphoreType.DMA,
            pltpu.SemaphoreType.DMA,
        ],
        compiler_params=_sc_params(),
    )
    def k(ta_hbm, tb_hbm, x_hbm, row_hbm, col_hbm, oa_hbm, ob_hbm, aux_hbm,
          idx_a, idx_b, buf_a, buf_b, xbuf, stage, sem_a, sem_b, sem_x):
        wid = lax.axis_index("s") * 2 + lax.axis_index("c")
        pltpu.async_copy(x_hbm, xbuf, sem_x).wait()

        @pl.loop(wid, nblk, step=NW)
        def _(b):
            base = b * EBLK
            pltpu.sync_copy(row_hbm.at[pl.ds(base, EBLK)], idx_a)
            pltpu.sync_copy(col_hbm.at[pl.ds(base, EBLK)], idx_b)
            cp_a = pltpu.async_copy(ta_hbm.at[idx_a], buf_a, sem_a)
            cp_b = pltpu.async_copy(tb_hbm.at[idx_b], buf_b, sem_b)
            for j in range(EBLK // L):
                ia3 = idx_a[pl.ds(j * L, L)] * 3
                ib3 = idx_b[pl.ds(j * L, L)] * 3
                rad = jnp.zeros((L,), F32)
                for d in range(3):
                    ds = jnp.full((L,), d, jnp.int32)
                    cd = (plsc.load_gather(xbuf, [ia3 + ds])
                          - plsc.load_gather(xbuf, [ib3 + ds]))
                    stage[d, pl.ds(j * L, L)] = cd
                    rad = rad + cd * cd
                stage[3, pl.ds(j * L, L)] = rad
            pltpu.sync_copy(stage, aux_hbm.at[pl.ds(b * 8, 8)])
            cp_a.wait()
            cp_b.wait()
            pltpu.sync_copy(buf_a, oa_hbm.at[pl.ds(base, EBLK)])
            pltpu.sync_copy(buf_b, ob_hbm.at[pl.ds(base, EBLK)])

    return k(tab_a, tab_b, xflat, row, col)


@jax.jit
def _sc_scatter(m, tail, row, init2):
    ec = m.shape[0]
    nblk = ec // EBLK

    @functools.partial(
        pl.kernel,
        out_type=jax.ShapeDtypeStruct((2, N, HID), F32),
        mesh=_sc_mesh(),
        scratch_types=[
            pltpu.VMEM((EBLK,), jnp.int32),
            pltpu.VMEM((EBLK, HID), F32),
            pltpu.VMEM_SHARED((N, HID), F32),
        ],
        compiler_params=_sc_params(),
    )
    def k(m_hbm, tail_hbm, row_hbm, z_hbm, out_hbm, idx, buf, acc):
        cid = lax.axis_index("c")
        sid = lax.axis_index("s")
        chunk = 80

        @pl.loop(sid, N // chunk, step=NSUB)
        def _(g):
            sl = pl.ds(g * chunk, chunk)
            pltpu.sync_copy(z_hbm.at[cid, sl], acc.at[sl])

        plsc.subcore_barrier()

        @pl.when(cid == 0)
        def _():
            @pl.loop(sid, nblk, step=NSUB)
            def _(b):
                base = b * EBLK
                pltpu.sync_copy(row_hbm.at[pl.ds(base, EBLK)], idx)
                pltpu.sync_copy(m_hbm.at[pl.ds(base, EBLK)], buf)
                pltpu.sync_copy(buf, acc.at[idx], add=True)

        @pl.when(cid == 1)
        def _():
            @pl.loop(sid, nblk, step=NSUB)
            def _(b):
                base = b * EBLK
                pltpu.sync_copy(row_hbm.at[pl.ds(base, EBLK)], idx)
                pltpu.sync_copy(tail_hbm.at[pl.ds(base, EBLK)], buf)
                pltpu.sync_copy(buf, acc.at[idx], add=True)

        plsc.subcore_barrier()

        @pl.loop(sid, N // chunk, step=NSUB)
        def _(g):
            sl = pl.ds(g * chunk, chunk)
            pltpu.sync_copy(acc.at[sl], out_hbm.at[cid, sl])

    return k(m, tail, row, init2)


def _full(shape):
    nd = len(shape)
    return pl.BlockSpec(shape, lambda *_: (0,) * nd)


def _init_tc(h, embw, embb, waw, wab, wbw):
    def body(h_ref, ew_ref, eb_ref, aw_ref, ab_ref, bw_ref,
             h0_ref, ta_ref, tb_ref):
        h0 = _dot(h_ref[...], ew_ref[...]) + eb_ref[...]
        h0_ref[...] = h0
        ta_ref[...] = _dot(h0, aw_ref[...]) + ab_ref[...]
        tb_ref[...] = _dot(h0, bw_ref[...])

    blk = pl.BlockSpec((NTBLK, HID), lambda i: (i, 0))
    return pl.pallas_call(
        body,
        grid=(N // NTBLK,),
        in_specs=[blk, _full((HID, HID)), _full((1, HID)),
                  _full((HID, HID)), _full((1, HID)), _full((HID, HID))],
        out_specs=[blk, blk, blk],
        out_shape=[jax.ShapeDtypeStruct((N, HID), F32)] * 3,
    )(h, embw, embb, waw, wab, wbw)


def _edge_tc(ga, gb, aux, e2w, e2b, c1w, c1b, c2r, wr):
    def body(ga_ref, gb_ref, aux_ref, e2w_ref, e2b_ref, c1w_ref, c1b_ref,
             c2r_ref, wr_ref, m_ref, tail_ref):
        a3 = aux_ref[...].reshape(SUBB, 8, EBLK)
        li = lax.broadcasted_iota(jnp.int32, (TBLK, EBLK), 1)
        ri = lax.broadcasted_iota(jnp.int32, (TBLK, EBLK), 0)
        dmask = (li == ri % EBLK).astype(F32)

        def col(d):
            b = jnp.broadcast_to(a3[:, d:d + 1, :],
                                 (SUBB, EBLK, EBLK)).reshape(TBLK, EBLK)
            return jnp.sum(b * dmask, axis=1, keepdims=True)

        cd0, cd1, cd2, radial = col(0), col(1), col(2), col(3)
        pre = ga_ref[...] + gb_ref[...] + radial * wr_ref[...]
        m = _silu(_dot_h(_silu(pre), e2w_ref[...]) + e2b_ref[...])
        t2 = _silu(_dot_h(m, c1w_ref[...]) + c1b_ref[...])
        t = jnp.sum(t2 * c2r_ref[...], axis=1, keepdims=True)
        m_ref[...] = m
        tail_ref[...] = jnp.concatenate(
            [cd0 * t, cd1 * t, cd2 * t, jnp.ones((TBLK, 1), F32),
             jnp.zeros((TBLK, HID - 4), F32)], axis=1)

    ec = ga.shape[0]
    eblk = pl.BlockSpec((TBLK, HID), lambda i: (i, 0))
    return pl.pallas_call(
        body,
        grid=(ec // TBLK,),
        in_specs=[
            eblk, eblk,
            pl.BlockSpec((SUBB * 8, EBLK), lambda i: (i, 0)),
            _full((HID, HID)), _full((1, HID)),
            _full((HID, HID)), _full((1, HID)),
            _full((1, HID)), _full((1, HID)),
        ],
        out_specs=[eblk, eblk],
        out_shape=[jax.ShapeDtypeStruct((ec, HID), F32)] * 2,
    )(ga, gb, aux, e2w, e2b, c1w, c1b, c2r, wr)


def _node_tc(h, x, parts, n1aw, n1bw, n1b, n2w, n2b, naw, nab, nbw):
    def body(h_ref, x_ref, p_ref, n1aw_ref, n1bw_ref, n1b_ref,
             n2w_ref, n2b_ref, naw_ref, nab_ref, nbw_ref,
             ho_ref, xo_ref, ta_ref, tb_ref):
        tlv = p_ref[1]
        tsum = tlv[:, :3]
        cnt = tlv[:, 3:4]
        xo_ref[...] = x_ref[...] + tsum / jnp.maximum(cnt, 1.0)
        agg = p_ref[0]
        h_in = h_ref[...]
        z = _silu(_dot(h_in, n1aw_ref[...]) + _dot(agg, n1bw_ref[...])
                  + n1b_ref[...])
        hn = h_in + _dot(z, n2w_ref[...]) + n2b_ref[...]
        ho_ref[...] = hn
        ta_ref[...] = _dot(hn, naw_ref[...]) + nab_ref[...]
        tb_ref[...] = _dot(hn, nbw_ref[...])

    blk = pl.BlockSpec((NTBLK, HID), lambda i: (i, 0))
    blk3 = pl.BlockSpec((NTBLK, 3), lambda i: (i, 0))
    pblk = pl.BlockSpec((2, NTBLK, HID), lambda i: (0, i, 0))
    return pl.pallas_call(
        body,
        grid=(N // NTBLK,),
        in_specs=[
            blk, blk3, pblk,
            _full((HID, HID)), _full((HID, HID)), _full((1, HID)),
            _full((HID, HID)), _full((1, HID)),
            _full((HID, HID)), _full((1, HID)), _full((HID, HID)),
        ],
        out_specs=[blk, blk3, blk, blk],
        out_shape=[
            jax.ShapeDtypeStruct((N, HID), F32),
            jax.ShapeDtypeStruct((N, 3), F32),
            jax.ShapeDtypeStruct((N, HID), F32),
            jax.ShapeDtypeStruct((N, HID), F32),
        ],
    )(h, x, parts, n1aw, n1bw, n1b, n2w, n2b, naw, nab, nbw)


def _head_tc(tab, ca2, l1w, l1b, l2w, l2b):
    ng = ca2.shape[0]

    def body(tab_ref, ca_ref, l1w_ref, l1b_ref, l2w_ref, l2b_ref, out_ref):
        iota = lax.broadcasted_iota(jnp.int32, (ng, N), 1)
        oh = (iota == ca_ref[...]).astype(F32)
        ch = _dot(oh, tab_ref[...])
        y = jnp.maximum(_dot(ch, l1w_ref[...]) + l1b_ref[...], 0.0)
        out_ref[...] = _dot(y, l2w_ref[...]) + l2b_ref[...]

    return pl.pallas_call(
        body,
        in_specs=[
            _full((N, HID)), _full((ng, 1)),
            _full((HID, 64)), _full((1, 64)),
            _full((64, HID)), _full((1, HID)),
        ],
        out_specs=_full((ng, HID)),
        out_shape=jax.ShapeDtypeStruct((ng, HID), F32),
    )(tab, ca2, l1w, l1b, l2w, l2b)


def kernel(h, x, edges, ca_idx, params):
    row = edges[0]
    col = edges[1]
    ech = E // 2
    rows = (row[:ech], row[ech:])
    cols = (col[:ech], col[ech:])
    zeros_2nh = jnp.zeros((2, N, HID), F32)
    zeros_hh = jnp.zeros((HID, HID), F32)

    def r1(v):
        return v.reshape(1, -1)

    lps = params["layers"]

    def proj_w(lp):
        e1w = lp["e1"]["w"]
        return e1w[:HID], r1(lp["e1"]["b"]), e1w[HID:2 * HID], r1(e1w[2 * HID])

    wa0, ab0, wb0, _ = proj_w(lps[0])
    hcur, ta, tb = _init_tc(h, params["emb_in"]["w"], r1(params["emb_in"]["b"]),
                            wa0, ab0, wb0)
    xcur = x
    for li, lp in enumerate(lps):
        _, _, _, wr = proj_w(lp)
        xflat = xcur.reshape(-1)
        gs = [_sc_gather(ta, tb, xflat, rows[c], cols[c]) for c in range(2)]
        parts = zeros_2nh
        for c in range(2):
            ga, gb, aux = gs[c]
            m, tail = _edge_tc(ga, gb, aux,
                               lp["e2"]["w"], r1(lp["e2"]["b"]),
                               lp["c1"]["w"], r1(lp["c1"]["b"]),
                               r1(lp["c2"]["w"]), wr)
            parts = _sc_scatter(m, tail, rows[c], parts)
        if li + 1 < len(lps):
            naw, nab, nbw, _ = proj_w(lps[li + 1])
        else:
            naw, nab, nbw = (params["emb_out"]["w"],
                             r1(params["emb_out"]["b"]), zeros_hh)
        n1w = lp["n1"]["w"]
        hcur, xcur, ta, tb = _node_tc(
            hcur, xcur, parts,
            n1w[:HID], n1w[HID:], r1(lp["n1"]["b"]),
            lp["n2"]["w"], r1(lp["n2"]["b"]),
            naw, nab, nbw)
    return _head_tc(ta, ca_idx.reshape(-1, 1).astype(jnp.int32),
                    params["mlp_l1"]["w"], r1(params["mlp_l1"]["b"]),
                    params["mlp_l2"]["w"], r1(params["mlp_l2"]["b"]))


# speedup vs baseline: 4.5679x; 1.1768x over previous
import dataclasses
import functools

import jax
import jax.numpy as jnp
from jax import lax
from jax.experimental import pallas as pl
from jax.experimental.pallas import tpu as pltpu
from jax.experimental.pallas import tpu_sc as plsc

N = 10000
E = 160000
HID = 128
EBLK = 128
NBLK = E // EBLK
NW = 32
NSUB = 16
L = 16
TBLK = 3200
SUBB = TBLK // EBLK
NTBLK = 1000
PREC = jax.lax.Precision.HIGHEST
F32 = jnp.float32


def _silu(v):
    return v / (1.0 + jnp.exp(-v))


def _dot(a, b):
    return jnp.dot(a, b, precision=PREC, preferred_element_type=F32)


def _dot_h(a, b):
    bf16 = jnp.bfloat16
    ah = a.astype(bf16)
    al = (a - ah.astype(F32)).astype(bf16)
    bh = b.astype(bf16)
    bl = (b - bh.astype(F32)).astype(bf16)

    def d(u, v):
        return jnp.dot(u, v, preferred_element_type=F32)

    return d(ah, bl) + d(al, bh) + d(ah, bh)


def _sc_params():
    cp = pltpu.CompilerParams()
    if "needs_layout_passes" in pltpu.CompilerParams.__dataclass_fields__:
        cp = dataclasses.replace(cp, needs_layout_passes=False)
    return cp


@functools.cache
def _sc_mesh():
    return plsc.VectorSubcoreMesh(core_axis_name="c", subcore_axis_name="s",
                                  num_cores=2, num_subcores=NSUB)


@jax.jit
def _sc_gather(tab_a, tab_b, xflat, row, col):
    ec = row.shape[0]
    nblk = ec // EBLK

    npair = nblk // 2

    @functools.partial(
        pl.kernel,
        out_type=(jax.ShapeDtypeStruct((ec, HID), F32),
                  jax.ShapeDtypeStruct((ec, HID), F32),
                  jax.ShapeDtypeStruct((nblk * 8, EBLK), F32)),
        mesh=_sc_mesh(),
        scratch_types=[
            pltpu.VMEM((2 * EBLK,), jnp.int32),
            pltpu.VMEM((2 * EBLK,), jnp.int32),
            pltpu.VMEM((2 * EBLK, HID), F32),
            pltpu.VMEM((2 * EBLK, HID), F32),
            pltpu.VMEM((3 * N,), F32),
            pltpu.VMEM((16, EBLK), F32),
            pltpu.SemaphoreType.DMA,
            pltpu.SemaphoreType.DMA,
            pltpu.SemaphoreType.DMA,
        ],
        compiler_params=_sc_params(),
    )
    def k(ta_hbm, tb_hbm, x_hbm, row_hbm, col_hbm, oa_hbm, ob_hbm, aux_hbm,
          idx_a, idx_b, buf_a, buf_b, xbuf, stage, sem_a, sem_b, sem_x):
        wid = lax.axis_index("s") * 2 + lax.axis_index("c")
        pltpu.async_copy(x_hbm, xbuf, sem_x).wait()

        def coords(nb):
            for j in range(nb * (EBLK // L)):
                ro = 8 * (j // (EBLK // L))
                lo = (j % (EBLK // L)) * L
                ia3 = idx_a[pl.ds(j * L, L)] * 3
                ib3 = idx_b[pl.ds(j * L, L)] * 3
                rad = jnp.zeros((L,), F32)
                for d in range(3):
                    ds = jnp.full((L,), d, jnp.int32)
                    cd = (plsc.load_gather(xbuf, [ia3 + ds])
                          - plsc.load_gather(xbuf, [ib3 + ds]))
                    stage[ro + d, pl.ds(lo, L)] = cd
                    rad = rad + cd * cd
                stage[ro + 3, pl.ds(lo, L)] = rad

        @pl.loop(wid, npair, step=NW)
        def _(p):
            base = p * 2 * EBLK
            pltpu.sync_copy(row_hbm.at[pl.ds(base, 2 * EBLK)], idx_a)
            pltpu.sync_copy(col_hbm.at[pl.ds(base, 2 * EBLK)], idx_b)
            cps = [
                pltpu.async_copy(ta_hbm.at[idx_a.at[pl.ds(0, EBLK)]],
                                 buf_a.at[pl.ds(0, EBLK)], sem_a),
                pltpu.async_copy(ta_hbm.at[idx_a.at[pl.ds(EBLK, EBLK)]],
                                 buf_a.at[pl.ds(EBLK, EBLK)], sem_a),
                pltpu.async_copy(tb_hbm.at[idx_b.at[pl.ds(0, EBLK)]],
                                 buf_b.at[pl.ds(0, EBLK)], sem_b),
                pltpu.async_copy(tb_hbm.at[idx_b.at[pl.ds(EBLK, EBLK)]],
                                 buf_b.at[pl.ds(EBLK, EBLK)], sem_b),
            ]
            coords(2)
            pltpu.sync_copy(stage, aux_hbm.at[pl.ds(p * 16, 16)])
            for cp in cps:
                cp.wait()
            pltpu.sync_copy(buf_a, oa_hbm.at[pl.ds(base, 2 * EBLK)])
            pltpu.sync_copy(buf_b, ob_hbm.at[pl.ds(base, 2 * EBLK)])

        @pl.loop(wid + 2 * npair, nblk, step=NW)
        def _(b):
            base = b * EBLK
            pltpu.sync_copy(row_hbm.at[pl.ds(base, EBLK)],
                            idx_a.at[pl.ds(0, EBLK)])
            pltpu.sync_copy(col_hbm.at[pl.ds(base, EBLK)],
                            idx_b.at[pl.ds(0, EBLK)])
            cp_a = pltpu.async_copy(ta_hbm.at[idx_a.at[pl.ds(0, EBLK)]],
                                    buf_a.at[pl.ds(0, EBLK)], sem_a)
            cp_b = pltpu.async_copy(tb_hbm.at[idx_b.at[pl.ds(0, EBLK)]],
                                    buf_b.at[pl.ds(0, EBLK)], sem_b)
            coords(1)
            pltpu.sync_copy(stage.at[pl.ds(0, 8)], aux_hbm.at[pl.ds(b * 8, 8)])
            cp_a.wait()
            cp_b.wait()
            pltpu.sync_copy(buf_a.at[pl.ds(0, EBLK)],
                            oa_hbm.at[pl.ds(base, EBLK)])
            pltpu.sync_copy(buf_b.at[pl.ds(0, EBLK)],
                            ob_hbm.at[pl.ds(base, EBLK)])

    return k(tab_a, tab_b, xflat, row, col)


@jax.jit
def _sc_scatter(m, tail, row, init2):
    ec = m.shape[0]
    nblk = ec // EBLK
    BB = 3
    ngrp = nblk // BB

    @functools.partial(
        pl.kernel,
        out_type=jax.ShapeDtypeStruct((2, N, HID), F32),
        mesh=_sc_mesh(),
        scratch_types=[
            pltpu.VMEM((BB * EBLK,), jnp.int32),
            pltpu.VMEM((BB * EBLK, HID), F32),
            pltpu.VMEM_SHARED((N, HID), F32),
        ],
        compiler_params=_sc_params(),
    )
    def k(m_hbm, tail_hbm, row_hbm, z_hbm, out_hbm, idx, buf, acc):
        cid = lax.axis_index("c")
        sid = lax.axis_index("s")
        chunk = 400

        @pl.loop(sid, N // chunk, step=NSUB)
        def _(g):
            sl = pl.ds(g * chunk, chunk)
            pltpu.sync_copy(z_hbm.at[cid, sl], acc.at[sl])

        plsc.subcore_barrier()

        def scat(src_hbm):
            @pl.loop(sid, ngrp, step=NSUB)
            def _(g):
                base = g * BB * EBLK
                pltpu.sync_copy(row_hbm.at[pl.ds(base, BB * EBLK)], idx)
                pltpu.sync_copy(src_hbm.at[pl.ds(base, BB * EBLK)], buf)
                for k in range(BB):
                    sl = pl.ds(k * EBLK, EBLK)
                    pltpu.sync_copy(buf.at[sl], acc.at[idx.at[sl]], add=True)

            @pl.loop(sid + BB * ngrp, nblk, step=NSUB)
            def _(b):
                base = b * EBLK
                sl = pl.ds(0, EBLK)
                pltpu.sync_copy(row_hbm.at[pl.ds(base, EBLK)], idx.at[sl])
                pltpu.sync_copy(src_hbm.at[pl.ds(base, EBLK)], buf.at[sl])
                pltpu.sync_copy(buf.at[sl], acc.at[idx.at[sl]], add=True)

        @pl.when(cid == 0)
        def _():
            scat(m_hbm)

        @pl.when(cid == 1)
        def _():
            scat(tail_hbm)

        plsc.subcore_barrier()

        @pl.loop(sid, N // chunk, step=NSUB)
        def _(g):
            sl = pl.ds(g * chunk, chunk)
            pltpu.sync_copy(acc.at[sl], out_hbm.at[cid, sl])

    return k(m, tail, row, init2)


def _full(shape):
    nd = len(shape)
    return pl.BlockSpec(shape, lambda *_: (0,) * nd)


def _init_tc(h, embw, embb, waw, wab, wbw):
    def body(h_ref, ew_ref, eb_ref, aw_ref, ab_ref, bw_ref,
             h0_ref, ta_ref, tb_ref):
        h0 = _dot(h_ref[...], ew_ref[...]) + eb_ref[...]
        h0_ref[...] = h0
        ta_ref[...] = _dot(h0, aw_ref[...]) + ab_ref[...]
        tb_ref[...] = _dot(h0, bw_ref[...])

    blk = pl.BlockSpec((NTBLK, HID), lambda i: (i, 0))
    return pl.pallas_call(
        body,
        grid=(N // NTBLK,),
        in_specs=[blk, _full((HID, HID)), _full((1, HID)),
                  _full((HID, HID)), _full((1, HID)), _full((HID, HID))],
        out_specs=[blk, blk, blk],
        out_shape=[jax.ShapeDtypeStruct((N, HID), F32)] * 3,
    )(h, embw, embb, waw, wab, wbw)


def _edge_tc(ga, gb, aux, e2w, e2b, c1w, c1b, c2r, wr):
    def body(ga_ref, gb_ref, aux_ref, e2w_ref, e2b_ref, c1w_ref, c1b_ref,
             c2r_ref, wr_ref, m_ref, tail_ref):
        a3 = aux_ref[...].reshape(SUBB, 8, EBLK)
        li = lax.broadcasted_iota(jnp.int32, (TBLK, EBLK), 1)
        ri = lax.broadcasted_iota(jnp.int32, (TBLK, EBLK), 0)
        dmask = (li == ri % EBLK).astype(F32)

        def col(d):
            b = jnp.broadcast_to(a3[:, d:d + 1, :],
                                 (SUBB, EBLK, EBLK)).reshape(TBLK, EBLK)
            return jnp.sum(b * dmask, axis=1, keepdims=True)

        cd0, cd1, cd2, radial = col(0), col(1), col(2), col(3)
        pre = ga_ref[...] + gb_ref[...] + radial * wr_ref[...]
        m = _silu(_dot_h(_silu(pre), e2w_ref[...]) + e2b_ref[...])
        t2 = _silu(_dot_h(m, c1w_ref[...]) + c1b_ref[...])
        t = jnp.sum(t2 * c2r_ref[...], axis=1, keepdims=True)
        m_ref[...] = m
        tail_ref[...] = jnp.concatenate(
            [cd0 * t, cd1 * t, cd2 * t, jnp.ones((TBLK, 1), F32),
             jnp.zeros((TBLK, HID - 4), F32)], axis=1)

    ec = ga.shape[0]
    eblk = pl.BlockSpec((TBLK, HID), lambda i: (i, 0))
    return pl.pallas_call(
        body,
        grid=(ec // TBLK,),
        in_specs=[
            eblk, eblk,
            pl.BlockSpec((SUBB * 8, EBLK), lambda i: (i, 0)),
            _full((HID, HID)), _full((1, HID)),
            _full((HID, HID)), _full((1, HID)),
            _full((1, HID)), _full((1, HID)),
        ],
        out_specs=[eblk, eblk],
        out_shape=[jax.ShapeDtypeStruct((ec, HID), F32)] * 2,
    )(ga, gb, aux, e2w, e2b, c1w, c1b, c2r, wr)


def _node_tc(h, x, parts, n1aw, n1bw, n1b, n2w, n2b, naw, nab, nbw):
    def body(h_ref, x_ref, p_ref, n1aw_ref, n1bw_ref, n1b_ref,
             n2w_ref, n2b_ref, naw_ref, nab_ref, nbw_ref,
             ho_ref, xo_ref, ta_ref, tb_ref):
        tlv = p_ref[1]
        tsum = tlv[:, :3]
        cnt = tlv[:, 3:4]
        xo_ref[...] = x_ref[...] + tsum / jnp.maximum(cnt, 1.0)
        agg = p_ref[0]
        h_in = h_ref[...]
        z = _silu(_dot(h_in, n1aw_ref[...]) + _dot(agg, n1bw_ref[...])
                  + n1b_ref[...])
        hn = h_in + _dot(z, n2w_ref[...]) + n2b_ref[...]
        ho_ref[...] = hn
        ta_ref[...] = _dot(hn, naw_ref[...]) + nab_ref[...]
        tb_ref[...] = _dot(hn, nbw_ref[...])

    blk = pl.BlockSpec((NTBLK, HID), lambda i: (i, 0))
    blk3 = pl.BlockSpec((NTBLK, 3), lambda i: (i, 0))
    pblk = pl.BlockSpec((2, NTBLK, HID), lambda i: (0, i, 0))
    return pl.pallas_call(
        body,
        grid=(N // NTBLK,),
        in_specs=[
            blk, blk3, pblk,
            _full((HID, HID)), _full((HID, HID)), _full((1, HID)),
            _full((HID, HID)), _full((1, HID)),
            _full((HID, HID)), _full((1, HID)), _full((HID, HID)),
        ],
        out_specs=[blk, blk3, blk, blk],
        out_shape=[
            jax.ShapeDtypeStruct((N, HID), F32),
            jax.ShapeDtypeStruct((N, 3), F32),
            jax.ShapeDtypeStruct((N, HID), F32),
            jax.ShapeDtypeStruct((N, HID), F32),
        ],
    )(h, x, parts, n1aw, n1bw, n1b, n2w, n2b, naw, nab, nbw)


def _head_tc(tab, ca2, l1w, l1b, l2w, l2b):
    ng = ca2.shape[0]

    def body(tab_ref, ca_ref, l1w_ref, l1b_ref, l2w_ref, l2b_ref, out_ref):
        iota = lax.broadcasted_iota(jnp.int32, (ng, N), 1)
        oh = (iota == ca_ref[...]).astype(F32)
        ch = _dot(oh, tab_ref[...])
        y = jnp.maximum(_dot(ch, l1w_ref[...]) + l1b_ref[...], 0.0)
        out_ref[...] = _dot(y, l2w_ref[...]) + l2b_ref[...]

    return pl.pallas_call(
        body,
        in_specs=[
            _full((N, HID)), _full((ng, 1)),
            _full((HID, 64)), _full((1, 64)),
            _full((64, HID)), _full((1, HID)),
        ],
        out_specs=_full((ng, HID)),
        out_shape=jax.ShapeDtypeStruct((ng, HID), F32),
    )(tab, ca2, l1w, l1b, l2w, l2b)


def kernel(h, x, edges, ca_idx, params):
    row = edges[0]
    col = edges[1]
    ech = E // 2
    rows = (row[:ech], row[ech:])
    cols = (col[:ech], col[ech:])
    zeros_2nh = jnp.zeros((2, N, HID), F32)
    zeros_hh = jnp.zeros((HID, HID), F32)

    def r1(v):
        return v.reshape(1, -1)

    lps = params["layers"]

    def proj_w(lp):
        e1w = lp["e1"]["w"]
        return e1w[:HID], r1(lp["e1"]["b"]), e1w[HID:2 * HID], r1(e1w[2 * HID])

    wa0, ab0, wb0, _ = proj_w(lps[0])
    hcur, ta, tb = _init_tc(h, params["emb_in"]["w"], r1(params["emb_in"]["b"]),
                            wa0, ab0, wb0)
    xcur = x
    for li, lp in enumerate(lps):
        _, _, _, wr = proj_w(lp)
        xflat = xcur.reshape(-1)
        gs = [_sc_gather(ta, tb, xflat, rows[c], cols[c]) for c in range(2)]
        parts = zeros_2nh
        for c in range(2):
            ga, gb, aux = gs[c]
            m, tail = _edge_tc(ga, gb, aux,
                               lp["e2"]["w"], r1(lp["e2"]["b"]),
                               lp["c1"]["w"], r1(lp["c1"]["b"]),
                               r1(lp["c2"]["w"]), wr)
            parts = _sc_scatter(m, tail, rows[c], parts)
        if li + 1 < len(lps):
            naw, nab, nbw, _ = proj_w(lps[li + 1])
        else:
            naw, nab, nbw = (params["emb_out"]["w"],
                             r1(params["emb_out"]["b"]), zeros_hh)
        n1w = lp["n1"]["w"]
        hcur, xcur, ta, tb = _node_tc(
            hcur, xcur, parts,
            n1w[:HID], n1w[HID:], r1(lp["n1"]["b"]),
            lp["n2"]["w"], r1(lp["n2"]["b"]),
            naw, nab, nbw)
    return _head_tc(ta, ca_idx.reshape(-1, 1).astype(jnp.int32),
                    params["mlp_l1"]["w"], r1(params["mlp_l1"]["b"]),
                    params["mlp_l2"]["w"], r1(params["mlp_l2"]["b"]))


# speedup vs baseline: 4.6561x; 1.0193x over previous
import dataclasses
import functools

import jax
import jax.numpy as jnp
from jax import lax
from jax.experimental import pallas as pl
from jax.experimental.pallas import tpu as pltpu
from jax.experimental.pallas import tpu_sc as plsc

N = 10000
E = 160000
HID = 128
EBLK = 128
NBLK = E // EBLK
NW = 32
NSUB = 16
L = 16
TBLK = 3200
SUBB = TBLK // EBLK
NTBLK = 1000
PREC = jax.lax.Precision.HIGHEST
F32 = jnp.float32


def _silu(v):
    return v / (1.0 + jnp.exp(-v))


def _dot(a, b):
    return jnp.dot(a, b, precision=PREC, preferred_element_type=F32)


def _dot_h(a, b):
    bf16 = jnp.bfloat16
    ah = a.astype(bf16)
    al = (a - ah.astype(F32)).astype(bf16)
    bh = b.astype(bf16)
    bl = (b - bh.astype(F32)).astype(bf16)

    def d(u, v):
        return jnp.dot(u, v, preferred_element_type=F32)

    return d(ah, bl) + d(al, bh) + d(ah, bh)


def _sc_params():
    cp = pltpu.CompilerParams()
    if "needs_layout_passes" in pltpu.CompilerParams.__dataclass_fields__:
        cp = dataclasses.replace(cp, needs_layout_passes=False)
    return cp


@functools.cache
def _sc_mesh():
    return plsc.VectorSubcoreMesh(core_axis_name="c", subcore_axis_name="s",
                                  num_cores=2, num_subcores=NSUB)


@jax.jit
def _sc_gather(tab_a, tab_b, xflat, rc, row, col):
    ec = row.shape[0]
    nblk = ec // EBLK

    npair = nblk // 2
    RCW = 4 * EBLK

    @functools.partial(
        pl.kernel,
        out_type=(jax.ShapeDtypeStruct((ec, HID), F32),
                  jax.ShapeDtypeStruct((ec, HID), F32),
                  jax.ShapeDtypeStruct((nblk * 8, EBLK), F32)),
        mesh=_sc_mesh(),
        scratch_types=[
            pltpu.VMEM((2, RCW), jnp.int32),
            pltpu.VMEM((2 * EBLK, HID), F32),
            pltpu.VMEM((2 * EBLK, HID), F32),
            pltpu.VMEM((3 * N,), F32),
            pltpu.VMEM((16, EBLK), F32),
            pltpu.SemaphoreType.DMA,
            pltpu.SemaphoreType.DMA,
            pltpu.SemaphoreType.DMA,
            pltpu.SemaphoreType.DMA,
        ],
        compiler_params=_sc_params(),
    )
    def k(ta_hbm, tb_hbm, x_hbm, rc_hbm, row_hbm, col_hbm,
          oa_hbm, ob_hbm, aux_hbm,
          idx, buf_a, buf_b, xbuf, stage, sem_a, sem_b, sem_x, sem_i):
        wid = lax.axis_index("s") * 2 + lax.axis_index("c")
        cp_x = pltpu.async_copy(x_hbm, xbuf, sem_x)
        pltpu.async_copy(rc_hbm.at[wid], idx.at[0], sem_i)
        cp_x.wait()

        def coords(nb, slot):
            for j in range(nb * (EBLK // L)):
                ro = 8 * (j // (EBLK // L))
                lo = (j % (EBLK // L)) * L
                ia3 = idx[slot, pl.ds(j * L, L)] * 3
                ib3 = idx[slot, pl.ds(2 * EBLK + j * L, L)] * 3
                rad = jnp.zeros((L,), F32)
                for d in range(3):
                    ds = jnp.full((L,), d, jnp.int32)
                    cd = (plsc.load_gather(xbuf, [ia3 + ds])
                          - plsc.load_gather(xbuf, [ib3 + ds]))
                    stage[ro + d, pl.ds(lo, L)] = cd
                    rad = rad + cd * cd
                stage[ro + 3, pl.ds(lo, L)] = rad

        @pl.loop(wid, npair, step=NW)
        def _(p):
            base = p * 2 * EBLK
            it = (p - wid) // NW
            slot = lax.rem(it, 2)
            pltpu.make_async_copy(rc_hbm.at[p], idx.at[slot], sem_i).wait()

            @pl.when(p + NW < npair)
            def _():
                pltpu.async_copy(rc_hbm.at[p + NW], idx.at[1 - slot], sem_i)

            cps = [
                pltpu.async_copy(ta_hbm.at[idx.at[slot, pl.ds(0, EBLK)]],
                                 buf_a.at[pl.ds(0, EBLK)], sem_a),
                pltpu.async_copy(ta_hbm.at[idx.at[slot, pl.ds(EBLK, EBLK)]],
                                 buf_a.at[pl.ds(EBLK, EBLK)], sem_a),
                pltpu.async_copy(
                    tb_hbm.at[idx.at[slot, pl.ds(2 * EBLK, EBLK)]],
                    buf_b.at[pl.ds(0, EBLK)], sem_b),
                pltpu.async_copy(
                    tb_hbm.at[idx.at[slot, pl.ds(3 * EBLK, EBLK)]],
                    buf_b.at[pl.ds(EBLK, EBLK)], sem_b),
            ]
            coords(2, slot)
            pltpu.sync_copy(stage, aux_hbm.at[pl.ds(p * 16, 16)])
            for cp in cps:
                cp.wait()
            pltpu.sync_copy(buf_a, oa_hbm.at[pl.ds(base, 2 * EBLK)])
            pltpu.sync_copy(buf_b, ob_hbm.at[pl.ds(base, 2 * EBLK)])

        @pl.loop(wid + 2 * npair, nblk, step=NW)
        def _(b):
            base = b * EBLK
            pltpu.sync_copy(row_hbm.at[pl.ds(base, EBLK)],
                            idx.at[0, pl.ds(0, EBLK)])
            pltpu.sync_copy(col_hbm.at[pl.ds(base, EBLK)],
                            idx.at[0, pl.ds(2 * EBLK, EBLK)])
            cp_a = pltpu.async_copy(ta_hbm.at[idx.at[0, pl.ds(0, EBLK)]],
                                    buf_a.at[pl.ds(0, EBLK)], sem_a)
            cp_b = pltpu.async_copy(
                tb_hbm.at[idx.at[0, pl.ds(2 * EBLK, EBLK)]],
                buf_b.at[pl.ds(0, EBLK)], sem_b)
            coords(1, 0)
            pltpu.sync_copy(stage.at[pl.ds(0, 8)], aux_hbm.at[pl.ds(b * 8, 8)])
            cp_a.wait()
            cp_b.wait()
            pltpu.sync_copy(buf_a.at[pl.ds(0, EBLK)],
                            oa_hbm.at[pl.ds(base, EBLK)])
            pltpu.sync_copy(buf_b.at[pl.ds(0, EBLK)],
                            ob_hbm.at[pl.ds(base, EBLK)])

    return k(tab_a, tab_b, xflat, rc, row, col)


@jax.jit
def _sc_scatter(m, tail, row, init2):
    ec = m.shape[0]
    nblk = ec // EBLK
    BB = 3
    ngrp = nblk // BB

    @functools.partial(
        pl.kernel,
        out_type=jax.ShapeDtypeStruct((2, N, HID), F32),
        mesh=_sc_mesh(),
        scratch_types=[
            pltpu.VMEM((BB * EBLK,), jnp.int32),
            pltpu.VMEM((BB * EBLK, HID), F32),
            pltpu.VMEM_SHARED((N, HID), F32),
        ],
        compiler_params=_sc_params(),
    )
    def k(m_hbm, tail_hbm, row_hbm, z_hbm, out_hbm, idx, buf, acc):
        cid = lax.axis_index("c")
        sid = lax.axis_index("s")
        chunk = 400

        @pl.loop(sid, N // chunk, step=NSUB)
        def _(g):
            sl = pl.ds(g * chunk, chunk)
            pltpu.sync_copy(z_hbm.at[cid, sl], acc.at[sl])

        plsc.subcore_barrier()

        def scat(src_hbm):
            @pl.loop(sid, ngrp, step=NSUB)
            def _(g):
                base = g * BB * EBLK
                pltpu.sync_copy(row_hbm.at[pl.ds(base, BB * EBLK)], idx)
                pltpu.sync_copy(src_hbm.at[pl.ds(base, BB * EBLK)], buf)
                for k in range(BB):
                    sl = pl.ds(k * EBLK, EBLK)
                    pltpu.sync_copy(buf.at[sl], acc.at[idx.at[sl]], add=True)

            @pl.loop(sid + BB * ngrp, nblk, step=NSUB)
            def _(b):
                base = b * EBLK
                sl = pl.ds(0, EBLK)
                pltpu.sync_copy(row_hbm.at[pl.ds(base, EBLK)], idx.at[sl])
                pltpu.sync_copy(src_hbm.at[pl.ds(base, EBLK)], buf.at[sl])
                pltpu.sync_copy(buf.at[sl], acc.at[idx.at[sl]], add=True)

        @pl.when(cid == 0)
        def _():
            scat(m_hbm)

        @pl.when(cid == 1)
        def _():
            scat(tail_hbm)

        plsc.subcore_barrier()

        @pl.loop(sid, N // chunk, step=NSUB)
        def _(g):
            sl = pl.ds(g * chunk, chunk)
            pltpu.sync_copy(acc.at[sl], out_hbm.at[cid, sl])

    return k(m, tail, row, init2)


def _full(shape):
    nd = len(shape)
    return pl.BlockSpec(shape, lambda *_: (0,) * nd)


def _init_tc(h, embw, embb, waw, wab, wbw):
    def body(h_ref, ew_ref, eb_ref, aw_ref, ab_ref, bw_ref,
             h0_ref, ta_ref, tb_ref):
        h0 = _dot(h_ref[...], ew_ref[...]) + eb_ref[...]
        h0_ref[...] = h0
        ta_ref[...] = _dot(h0, aw_ref[...]) + ab_ref[...]
        tb_ref[...] = _dot(h0, bw_ref[...])

    blk = pl.BlockSpec((NTBLK, HID), lambda i: (i, 0))
    return pl.pallas_call(
        body,
        grid=(N // NTBLK,),
        in_specs=[blk, _full((HID, HID)), _full((1, HID)),
                  _full((HID, HID)), _full((1, HID)), _full((HID, HID))],
        out_specs=[blk, blk, blk],
        out_shape=[jax.ShapeDtypeStruct((N, HID), F32)] * 3,
    )(h, embw, embb, waw, wab, wbw)


def _edge_tc(ga, gb, aux, e2w, e2b, c1w, c1b, c2r, wr):
    def body(ga_ref, gb_ref, aux_ref, e2w_ref, e2b_ref, c1w_ref, c1b_ref,
             c2r_ref, wr_ref, m_ref, tail_ref):
        a3 = aux_ref[...].reshape(SUBB, 8, EBLK)
        li = lax.broadcasted_iota(jnp.int32, (TBLK, EBLK), 1)
        ri = lax.broadcasted_iota(jnp.int32, (TBLK, EBLK), 0)
        dmask = (li == ri % EBLK).astype(F32)

        def col(d):
            b = jnp.broadcast_to(a3[:, d:d + 1, :],
                                 (SUBB, EBLK, EBLK)).reshape(TBLK, EBLK)
            return jnp.sum(b * dmask, axis=1, keepdims=True)

        cd0, cd1, cd2, radial = col(0), col(1), col(2), col(3)
        pre = ga_ref[...] + gb_ref[...] + radial * wr_ref[...]
        m = _silu(_dot_h(_silu(pre), e2w_ref[...]) + e2b_ref[...])
        t2 = _silu(_dot_h(m, c1w_ref[...]) + c1b_ref[...])
        t = jnp.sum(t2 * c2r_ref[...], axis=1, keepdims=True)
        m_ref[...] = m
        tail_ref[...] = jnp.concatenate(
            [cd0 * t, cd1 * t, cd2 * t, jnp.ones((TBLK, 1), F32),
             jnp.zeros((TBLK, HID - 4), F32)], axis=1)

    ec = ga.shape[0]
    eblk = pl.BlockSpec((TBLK, HID), lambda i: (i, 0))
    return pl.pallas_call(
        body,
        grid=(ec // TBLK,),
        in_specs=[
            eblk, eblk,
            pl.BlockSpec((SUBB * 8, EBLK), lambda i: (i, 0)),
            _full((HID, HID)), _full((1, HID)),
            _full((HID, HID)), _full((1, HID)),
            _full((1, HID)), _full((1, HID)),
        ],
        out_specs=[eblk, eblk],
        out_shape=[jax.ShapeDtypeStruct((ec, HID), F32)] * 2,
    )(ga, gb, aux, e2w, e2b, c1w, c1b, c2r, wr)


def _node_tc(h, x, parts, n1aw, n1bw, n1b, n2w, n2b, naw, nab, nbw):
    def body(h_ref, x_ref, p_ref, n1aw_ref, n1bw_ref, n1b_ref,
             n2w_ref, n2b_ref, naw_ref, nab_ref, nbw_ref,
             ho_ref, xo_ref, ta_ref, tb_ref):
        tlv = p_ref[1]
        tsum = tlv[:, :3]
        cnt = tlv[:, 3:4]
        xo_ref[...] = x_ref[...] + tsum / jnp.maximum(cnt, 1.0)
        agg = p_ref[0]
        h_in = h_ref[...]
        z = _silu(_dot(h_in, n1aw_ref[...]) + _dot(agg, n1bw_ref[...])
                  + n1b_ref[...])
        hn = h_in + _dot(z, n2w_ref[...]) + n2b_ref[...]
        ho_ref[...] = hn
        ta_ref[...] = _dot(hn, naw_ref[...]) + nab_ref[...]
        tb_ref[...] = _dot(hn, nbw_ref[...])

    blk = pl.BlockSpec((NTBLK, HID), lambda i: (i, 0))
    blk3 = pl.BlockSpec((NTBLK, 3), lambda i: (i, 0))
    pblk = pl.BlockSpec((2, NTBLK, HID), lambda i: (0, i, 0))
    return pl.pallas_call(
        body,
        grid=(N // NTBLK,),
        in_specs=[
            blk, blk3, pblk,
            _full((HID, HID)), _full((HID, HID)), _full((1, HID)),
            _full((HID, HID)), _full((1, HID)),
            _full((HID, HID)), _full((1, HID)), _full((HID, HID)),
        ],
        out_specs=[blk, blk3, blk, blk],
        out_shape=[
            jax.ShapeDtypeStruct((N, HID), F32),
            jax.ShapeDtypeStruct((N, 3), F32),
            jax.ShapeDtypeStruct((N, HID), F32),
            jax.ShapeDtypeStruct((N, HID), F32),
        ],
    )(h, x, parts, n1aw, n1bw, n1b, n2w, n2b, naw, nab, nbw)


def _head_tc(tab, ca2, l1w, l1b, l2w, l2b):
    ng = ca2.shape[0]

    def body(tab_ref, ca_ref, l1w_ref, l1b_ref, l2w_ref, l2b_ref, out_ref):
        iota = lax.broadcasted_iota(jnp.int32, (ng, N), 1)
        oh = (iota == ca_ref[...]).astype(F32)
        ch = _dot(oh, tab_ref[...])
        y = jnp.maximum(_dot(ch, l1w_ref[...]) + l1b_ref[...], 0.0)
        out_ref[...] = _dot(y, l2w_ref[...]) + l2b_ref[...]

    return pl.pallas_call(
        body,
        in_specs=[
            _full((N, HID)), _full((ng, 1)),
            _full((HID, 64)), _full((1, 64)),
            _full((64, HID)), _full((1, HID)),
        ],
        out_specs=_full((ng, HID)),
        out_shape=jax.ShapeDtypeStruct((ng, HID), F32),
    )(tab, ca2, l1w, l1b, l2w, l2b)


def kernel(h, x, edges, ca_idx, params):
    row = edges[0]
    col = edges[1]
    ech = E // 2
    rows = (row[:ech], row[ech:])
    cols = (col[:ech], col[ech:])

    def pack_rc(r, c):
        npair = (r.shape[0] // EBLK) // 2
        pe = npair * 2 * EBLK
        return jnp.concatenate([r[:pe].reshape(npair, 2 * EBLK),
                                c[:pe].reshape(npair, 2 * EBLK)], axis=1)

    rcs = (pack_rc(rows[0], cols[0]), pack_rc(rows[1], cols[1]))
    zeros_2nh = jnp.zeros((2, N, HID), F32)
    zeros_hh = jnp.zeros((HID, HID), F32)

    def r1(v):
        return v.reshape(1, -1)

    lps = params["layers"]

    def proj_w(lp):
        e1w = lp["e1"]["w"]
        return e1w[:HID], r1(lp["e1"]["b"]), e1w[HID:2 * HID], r1(e1w[2 * HID])

    wa0, ab0, wb0, _ = proj_w(lps[0])
    hcur, ta, tb = _init_tc(h, params["emb_in"]["w"], r1(params["emb_in"]["b"]),
                            wa0, ab0, wb0)
    xcur = x
    for li, lp in enumerate(lps):
        _, _, _, wr = proj_w(lp)
        xflat = xcur.reshape(-1)
        gs = [_sc_gather(ta, tb, xflat, rcs[c], rows[c], cols[c])
              for c in range(2)]
        parts = zeros_2nh
        for c in range(2):
            ga, gb, aux = gs[c]
            m, tail = _edge_tc(ga, gb, aux,
                               lp["e2"]["w"], r1(lp["e2"]["b"]),
                               lp["c1"]["w"], r1(lp["c1"]["b"]),
                               r1(lp["c2"]["w"]), wr)
            parts = _sc_scatter(m, tail, rows[c], parts)
        if li + 1 < len(lps):
            naw, nab, nbw, _ = proj_w(lps[li + 1])
        else:
            naw, nab, nbw = (params["emb_out"]["w"],
                             r1(params["emb_out"]["b"]), zeros_hh)
        n1w = lp["n1"]["w"]
        hcur, xcur, ta, tb = _node_tc(
            hcur, xcur, parts,
            n1w[:HID], n1w[HID:], r1(lp["n1"]["b"]),
            lp["n2"]["w"], r1(lp["n2"]["b"]),
            naw, nab, nbw)
    return _head_tc(ta, ca_idx.reshape(-1, 1).astype(jnp.int32),
                    params["mlp_l1"]["w"], r1(params["mlp_l1"]["b"]),
                    params["mlp_l2"]["w"], r1(params["mlp_l2"]["b"]))
